# baseline (speedup 1.0000x reference)
import jax
import jax.numpy as jnp
from jax.experimental import pallas as pl
from jax.experimental.pallas import tpu as pltpu

N = 10000
E = 256
C = 256
HEADS = 8
TILE = 1000
GRID = N // TILE
NCP = 128

f32 = jnp.float32
_ARB = pltpu.CompilerParams(dimension_semantics=("arbitrary",))


def _dot(a, b):
    return jax.lax.dot_general(a, b, (((1,), (0,)), ((), ())),
                               preferred_element_type=f32)


def _dot_tn(a, b):
    return jax.lax.dot_general(a, b, (((0,), (0,)), ((), ())),
                               preferred_element_type=f32)


def _dot_nt(a, b):
    return jax.lax.dot_general(a, b, (((1,), (1,)), ((), ())),
                               preferred_element_type=f32)


def _p1(x_ref, h_ref, w_ref, b_ref, s1_ref, de_ref):
    i = pl.program_id(0)

    @pl.when(i == 0)
    def _():
        s1_ref[...] = jnp.zeros_like(s1_ref)
        de_ref[...] = jnp.zeros_like(de_ref)

    h = h_ref[...]
    xw = _dot(x_ref[...], w_ref[...]) + b_ref[...]
    s1_ref[...] += _dot_tn(h, xw)
    de_ref[...] += jnp.sum(h, axis=0, keepdims=True)


def _p2(h_ref, y_ref, x1_ref, ze_ref):
    i = pl.program_id(0)

    @pl.when(i == 0)
    def _():
        ze_ref[...] = jnp.zeros_like(ze_ref)

    h = h_ref[...]
    dv = jnp.maximum(jnp.sum(h, axis=1, keepdims=True), 1.0)
    x1 = jnp.maximum(_dot(h, y_ref[...]) / dv, 0.0)
    x1_ref[...] = x1
    ze_ref[...] += _dot_tn(h, x1)


def _attn_kl(x1, ze, att_ref):
    acc = jnp.zeros(x1.shape[:1] + (E,), f32)
    for hh in range(HEADS):
        a2 = att_ref[hh:hh + 1, :]
        num = _dot_nt(x1, ze * a2)
        xn = jnp.sqrt(jnp.sum(x1 * x1 * a2, axis=1, keepdims=True))
        zn = jnp.sqrt(jnp.sum(ze * ze * a2, axis=1)).reshape(1, E)
        acc += num / jnp.maximum(xn * zn, 1e-6)
    a = acc * (1.0 / HEADS)
    ac = jnp.clip(a, 1e-6, 1.0 - 1e-6)
    return ac * jnp.log(ac * 2.0) + (1.0 - ac) * jnp.log((1.0 - ac) * 2.0)


def _p3(x1_ref, h_ref, ze_ref, att_ref, w11_ref, b11_ref, w2_ref, b2_ref,
        kl_ref, t1_ref, s2_ref):
    i = pl.program_id(0)

    @pl.when(i == 0)
    def _():
        kl_ref[...] = jnp.zeros_like(kl_ref)
        t1_ref[...] = jnp.zeros_like(t1_ref)
        s2_ref[...] = jnp.zeros_like(s2_ref)

    x1 = x1_ref[...]
    h = h_ref[...]
    kl = _attn_kl(x1, ze_ref[...], att_ref)
    kl_ref[...] += jnp.sum(kl, axis=0, keepdims=True)
    t1_ref[...] += _dot_tn(h, _dot(x1, w11_ref[...]) + b11_ref[...])
    s2_ref[...] += _dot_tn(h, _dot(x1, w2_ref[...]) + b2_ref[...])


def _p4(h_ref, y2_ref, u1_ref, x2_ref, y1_ref, ze2_ref):
    i = pl.program_id(0)

    @pl.when(i == 0)
    def _():
        ze2_ref[...] = jnp.zeros_like(ze2_ref)

    h = h_ref[...]
    dv = jnp.maximum(jnp.sum(h, axis=1, keepdims=True), 1.0)
    x2 = jnp.maximum(_dot(h, y2_ref[...]) / dv, 0.0)
    x2_ref[...] = x2
    y1_ref[...] = _dot(h, u1_ref[...]) / dv
    ze2_ref[...] += _dot_tn(h, x2)


def _p5(x2_ref, h_ref, ze_ref, att_ref, w21_ref, b21_ref, kl_ref, t2_ref):
    i = pl.program_id(0)

    @pl.when(i == 0)
    def _():
        kl_ref[...] = jnp.zeros_like(kl_ref)
        t2_ref[...] = jnp.zeros_like(t2_ref)

    x2 = x2_ref[...]
    h = h_ref[...]
    kl = _attn_kl(x2, ze_ref[...], att_ref)
    kl_ref[...] += jnp.sum(kl, axis=0, keepdims=True)
    t2_ref[...] += _dot_tn(h, _dot(x2, w21_ref[...]) + b21_ref[...])


def _p6(h_ref, u2_ref, y2_ref):
    h = h_ref[...]
    dv = jnp.maximum(jnp.sum(h, axis=1, keepdims=True), 1.0)
    y2_ref[...] = _dot(h, u2_ref[...]) / dv


def _row_spec(cols):
    return pl.BlockSpec((TILE, cols), lambda i: (i, 0))


def _full_spec(rows, cols):
    return pl.BlockSpec((rows, cols), lambda i: (0, 0))


def kernel(x, H, W1, b1, att1, W11, b11, W2, b2, att2, W21, b21):
    b1r = b1.reshape(1, C)
    b2r = b2.reshape(1, C)
    att1sq = att1 * att1
    att2sq = att2 * att2
    w11p = jnp.zeros((C, NCP), f32).at[:, :3].set(W11)
    b11p = jnp.zeros((1, NCP), f32).at[0, :3].set(b11)
    w21p = jnp.zeros((C, NCP), f32).at[:, :3].set(W21)
    b21p = jnp.zeros((1, NCP), f32).at[0, :3].set(b21)

    s1, de = pl.pallas_call(
        _p1,
        grid=(GRID,),
        in_specs=[_row_spec(C), _row_spec(E), _full_spec(C, C),
                  _full_spec(1, C)],
        out_specs=[_full_spec(E, C), _full_spec(1, E)],
        out_shape=[jax.ShapeDtypeStruct((E, C), f32),
                   jax.ShapeDtypeStruct((1, E), f32)],
        compiler_params=_ARB,
    )(x, H, W1, b1r)

    de_col = jnp.maximum(de, 1.0).reshape(E, 1)

    x1, ze1 = pl.pallas_call(
        _p2,
        grid=(GRID,),
        in_specs=[_row_spec(E), _full_spec(E, C)],
        out_specs=[_row_spec(C), _full_spec(E, C)],
        out_shape=[jax.ShapeDtypeStruct((N, C), f32),
                   jax.ShapeDtypeStruct((E, C), f32)],
        compiler_params=_ARB,
    )(H, s1 / de_col)

    klv1, t1, s2 = pl.pallas_call(
        _p3,
        grid=(GRID,),
        in_specs=[_row_spec(C), _row_spec(E), _full_spec(E, C),
                  _full_spec(HEADS, C), _full_spec(C, NCP), _full_spec(1, NCP),
                  _full_spec(C, C), _full_spec(1, C)],
        out_specs=[_full_spec(1, E), _full_spec(E, NCP), _full_spec(E, C)],
        out_shape=[jax.ShapeDtypeStruct((1, E), f32),
                   jax.ShapeDtypeStruct((E, NCP), f32),
                   jax.ShapeDtypeStruct((E, C), f32)],
        compiler_params=_ARB,
    )(x1, H, ze1, att1sq, w11p, b11p, W2, b2r)

    x2, y1p, ze2 = pl.pallas_call(
        _p4,
        grid=(GRID,),
        in_specs=[_row_spec(E), _full_spec(E, C), _full_spec(E, NCP)],
        out_specs=[_row_spec(C), _row_spec(NCP), _full_spec(E, C)],
        out_shape=[jax.ShapeDtypeStruct((N, C), f32),
                   jax.ShapeDtypeStruct((N, NCP), f32),
                   jax.ShapeDtypeStruct((E, C), f32)],
        compiler_params=_ARB,
    )(H, s2 / de_col, t1 / de_col)

    klv2, t2 = pl.pallas_call(
        _p5,
        grid=(GRID,),
        in_specs=[_row_spec(C), _row_spec(E), _full_spec(E, C),
                  _full_spec(HEADS, C), _full_spec(C, NCP), _full_spec(1, NCP)],
        out_specs=[_full_spec(1, E), _full_spec(E, NCP)],
        out_shape=[jax.ShapeDtypeStruct((1, E), f32),
                   jax.ShapeDtypeStruct((E, NCP), f32)],
        compiler_params=_ARB,
    )(x2, H, ze2, att2sq, w21p, b21p)

    y2p = pl.pallas_call(
        _p6,
        grid=(GRID,),
        in_specs=[_row_spec(E), _full_spec(E, NCP)],
        out_specs=_row_spec(NCP),
        out_shape=jax.ShapeDtypeStruct((N, NCP), f32),
        compiler_params=_ARB,
    )(H, t2 / de_col)

    kl1 = jnp.sum(klv1) / N
    kl2 = jnp.sum(klv2) / N
    return (y1p[:, :3], y2p[:, :3], (kl1 + kl2) * 0.5)


# speedup vs baseline: 1.2895x; 1.2895x over previous
import jax
import jax.numpy as jnp
from jax.experimental import pallas as pl
from jax.experimental.pallas import tpu as pltpu

N = 10000
E = 256
C = 256
HEADS = 8
TILE = 1000
GRID = N // TILE
NCP = 128

f32 = jnp.float32
_ARB = pltpu.CompilerParams(dimension_semantics=("arbitrary",))


def _dot(a, b):
    return jax.lax.dot_general(a, b, (((1,), (0,)), ((), ())),
                               preferred_element_type=f32)


def _dot_tn(a, b):
    return jax.lax.dot_general(a, b, (((0,), (0,)), ((), ())),
                               preferred_element_type=f32)


def _dot_nt(a, b):
    return jax.lax.dot_general(a, b, (((1,), (1,)), ((), ())),
                               preferred_element_type=f32)


def _p1(x_ref, h_ref, w_ref, b_ref, s1_ref, de_ref):
    i = pl.program_id(0)

    @pl.when(i == 0)
    def _():
        s1_ref[...] = jnp.zeros_like(s1_ref)
        de_ref[...] = jnp.zeros_like(de_ref)

    h = h_ref[...]
    xw = _dot(x_ref[...], w_ref[...]) + b_ref[...]
    s1_ref[...] += _dot_tn(h, xw)
    de_ref[...] += jnp.sum(h, axis=0, keepdims=True)


def _p2(h_ref, y_ref, x1_ref, ze_ref):
    i = pl.program_id(0)

    @pl.when(i == 0)
    def _():
        ze_ref[...] = jnp.zeros_like(ze_ref)

    h = h_ref[...]
    dv = jnp.maximum(jnp.sum(h, axis=1, keepdims=True), 1.0)
    x1 = jnp.maximum(_dot(h, y_ref[...]) / dv, 0.0)
    x1_ref[...] = x1
    ze_ref[...] += _dot_tn(h, x1)


def _attn_kl(x1, ze, attp_ref):
    attp = attp_ref[...]
    xn2 = _dot_nt(x1 * x1, attp)
    zn2 = _dot_nt(ze * ze, attp)
    inv_xn = jax.lax.rsqrt(jnp.maximum(xn2, 1e-24)) * (1.0 / HEADS)
    inv_zn = jax.lax.rsqrt(jnp.maximum(zn2, 1e-24))
    acc = jnp.zeros(x1.shape[:1] + (E,), f32)
    for hh in range(HEADS):
        x1h = (x1 * inv_xn[:, hh:hh + 1]).astype(jnp.bfloat16)
        zth = (ze * attp[hh:hh + 1, :] * inv_zn[:, hh:hh + 1]
               ).astype(jnp.bfloat16)
        acc += _dot_nt(x1h, zth)
    ac = jnp.minimum(jnp.maximum(acc, 1e-6), 1.0 - 1e-6)
    return ac * jnp.log(ac * 2.0) + (1.0 - ac) * jnp.log((1.0 - ac) * 2.0)


def _p3(x1_ref, h_ref, ze_ref, att_ref, w11_ref, b11_ref, w2_ref, b2_ref,
        kl_ref, t1_ref, s2_ref):
    i = pl.program_id(0)

    @pl.when(i == 0)
    def _():
        kl_ref[...] = jnp.zeros_like(kl_ref)
        t1_ref[...] = jnp.zeros_like(t1_ref)
        s2_ref[...] = jnp.zeros_like(s2_ref)

    x1 = x1_ref[...]
    h = h_ref[...]
    kl = _attn_kl(x1, ze_ref[...], att_ref)
    kl_ref[...] += jnp.sum(kl, axis=0, keepdims=True)
    t1_ref[...] += _dot_tn(h, _dot(x1, w11_ref[...]) + b11_ref[...])
    s2_ref[...] += _dot_tn(h, _dot(x1, w2_ref[...]) + b2_ref[...])


def _p4(h_ref, y2_ref, u1_ref, x2_ref, y1_ref, ze2_ref):
    i = pl.program_id(0)

    @pl.when(i == 0)
    def _():
        ze2_ref[...] = jnp.zeros_like(ze2_ref)

    h = h_ref[...]
    dv = jnp.maximum(jnp.sum(h, axis=1, keepdims=True), 1.0)
    x2 = jnp.maximum(_dot(h, y2_ref[...]) / dv, 0.0)
    x2_ref[...] = x2
    y1_ref[...] = _dot(h, u1_ref[...]) / dv
    ze2_ref[...] += _dot_tn(h, x2)


def _p5(x2_ref, h_ref, ze_ref, att_ref, w21_ref, b21_ref, kl_ref, t2_ref):
    i = pl.program_id(0)

    @pl.when(i == 0)
    def _():
        kl_ref[...] = jnp.zeros_like(kl_ref)
        t2_ref[...] = jnp.zeros_like(t2_ref)

    x2 = x2_ref[...]
    h = h_ref[...]
    kl = _attn_kl(x2, ze_ref[...], att_ref)
    kl_ref[...] += jnp.sum(kl, axis=0, keepdims=True)
    t2_ref[...] += _dot_tn(h, _dot(x2, w21_ref[...]) + b21_ref[...])


def _p6(h_ref, u2_ref, y2_ref):
    h = h_ref[...]
    dv = jnp.maximum(jnp.sum(h, axis=1, keepdims=True), 1.0)
    y2_ref[...] = _dot(h, u2_ref[...]) / dv


def _row_spec(cols):
    return pl.BlockSpec((TILE, cols), lambda i: (i, 0))


def _full_spec(rows, cols):
    return pl.BlockSpec((rows, cols), lambda i: (0, 0))


def kernel(x, H, W1, b1, att1, W11, b11, W2, b2, att2, W21, b21):
    b1r = b1.reshape(1, C)
    b2r = b2.reshape(1, C)
    att1sq = jnp.zeros((NCP, C), f32).at[:HEADS].set(att1 * att1)
    att2sq = jnp.zeros((NCP, C), f32).at[:HEADS].set(att2 * att2)
    w11p = jnp.zeros((C, NCP), f32).at[:, :3].set(W11)
    b11p = jnp.zeros((1, NCP), f32).at[0, :3].set(b11)
    w21p = jnp.zeros((C, NCP), f32).at[:, :3].set(W21)
    b21p = jnp.zeros((1, NCP), f32).at[0, :3].set(b21)

    s1, de = pl.pallas_call(
        _p1,
        grid=(GRID,),
        in_specs=[_row_spec(C), _row_spec(E), _full_spec(C, C),
                  _full_spec(1, C)],
        out_specs=[_full_spec(E, C), _full_spec(1, E)],
        out_shape=[jax.ShapeDtypeStruct((E, C), f32),
                   jax.ShapeDtypeStruct((1, E), f32)],
        compiler_params=_ARB,
    )(x, H, W1, b1r)

    de_col = jnp.maximum(de, 1.0).reshape(E, 1)

    x1, ze1 = pl.pallas_call(
        _p2,
        grid=(GRID,),
        in_specs=[_row_spec(E), _full_spec(E, C)],
        out_specs=[_row_spec(C), _full_spec(E, C)],
        out_shape=[jax.ShapeDtypeStruct((N, C), f32),
                   jax.ShapeDtypeStruct((E, C), f32)],
        compiler_params=_ARB,
    )(H, s1 / de_col)

    klv1, t1, s2 = pl.pallas_call(
        _p3,
        grid=(GRID,),
        in_specs=[_row_spec(C), _row_spec(E), _full_spec(E, C),
                  _full_spec(NCP, C), _full_spec(C, NCP), _full_spec(1, NCP),
                  _full_spec(C, C), _full_spec(1, C)],
        out_specs=[_full_spec(1, E), _full_spec(E, NCP), _full_spec(E, C)],
        out_shape=[jax.ShapeDtypeStruct((1, E), f32),
                   jax.ShapeDtypeStruct((E, NCP), f32),
                   jax.ShapeDtypeStruct((E, C), f32)],
        compiler_params=_ARB,
    )(x1, H, ze1, att1sq, w11p, b11p, W2, b2r)

    x2, y1p, ze2 = pl.pallas_call(
        _p4,
        grid=(GRID,),
        in_specs=[_row_spec(E), _full_spec(E, C), _full_spec(E, NCP)],
        out_specs=[_row_spec(C), _row_spec(NCP), _full_spec(E, C)],
        out_shape=[jax.ShapeDtypeStruct((N, C), f32),
                   jax.ShapeDtypeStruct((N, NCP), f32),
                   jax.ShapeDtypeStruct((E, C), f32)],
        compiler_params=_ARB,
    )(H, s2 / de_col, t1 / de_col)

    klv2, t2 = pl.pallas_call(
        _p5,
        grid=(GRID,),
        in_specs=[_row_spec(C), _row_spec(E), _full_spec(E, C),
                  _full_spec(NCP, C), _full_spec(C, NCP), _full_spec(1, NCP)],
        out_specs=[_full_spec(1, E), _full_spec(E, NCP)],
        out_shape=[jax.ShapeDtypeStruct((1, E), f32),
                   jax.ShapeDtypeStruct((E, NCP), f32)],
        compiler_params=_ARB,
    )(x2, H, ze2, att2sq, w21p, b21p)

    y2p = pl.pallas_call(
        _p6,
        grid=(GRID,),
        in_specs=[_row_spec(E), _full_spec(E, NCP)],
        out_specs=_row_spec(NCP),
        out_shape=jax.ShapeDtypeStruct((N, NCP), f32),
        compiler_params=_ARB,
    )(H, t2 / de_col)

    kl1 = jnp.sum(klv1) / N
    kl2 = jnp.sum(klv2) / N
    return (y1p[:, :3], y2p[:, :3], (kl1 + kl2) * 0.5)


# speedup vs baseline: 1.3208x; 1.0242x over previous
import jax
import jax.numpy as jnp
from jax.experimental import pallas as pl
from jax.experimental.pallas import tpu as pltpu

N = 10000
E = 256
C = 256
HEADS = 8
TILE = 1000
GRID = N // TILE
NCP = 128

f32 = jnp.float32
bf16 = jnp.bfloat16
_ARB = pltpu.CompilerParams(dimension_semantics=("arbitrary",))


def _dot(a, b):
    return jax.lax.dot_general(a, b, (((1,), (0,)), ((), ())),
                               preferred_element_type=f32)


def _dot_tn(a, b):
    return jax.lax.dot_general(a, b, (((0,), (0,)), ((), ())),
                               preferred_element_type=f32)


def _dot_nt(a, b):
    return jax.lax.dot_general(a, b, (((1,), (1,)), ((), ())),
                               preferred_element_type=f32)


def _p1(x_ref, h_ref, w_ref, b_ref, s1_ref, de_ref, hb_ref):
    i = pl.program_id(0)

    @pl.when(i == 0)
    def _():
        s1_ref[...] = jnp.zeros_like(s1_ref)
        de_ref[...] = jnp.zeros_like(de_ref)

    h = h_ref[...]
    hb = h.astype(bf16)
    hb_ref[...] = hb
    xw = _dot(x_ref[...].astype(bf16), w_ref[...]) + b_ref[...]
    s1_ref[...] += _dot_tn(hb, xw.astype(bf16))
    de_ref[...] += jnp.sum(h, axis=0, keepdims=True)


def _p2(hb_ref, y_ref, x1_ref, ze_ref):
    i = pl.program_id(0)

    @pl.when(i == 0)
    def _():
        ze_ref[...] = jnp.zeros_like(ze_ref)

    hb = hb_ref[...]
    dv = jnp.maximum(jnp.sum(hb.astype(f32), axis=1, keepdims=True), 1.0)
    x1 = jnp.maximum(_dot(hb, y_ref[...]) / dv, 0.0).astype(bf16)
    x1_ref[...] = x1
    ze_ref[...] += _dot_tn(hb, x1)


def _attn_kl(x1f, ze, attp_ref):
    attp = attp_ref[...]
    attb = attp.astype(bf16)
    xn2 = _dot_nt((x1f * x1f).astype(bf16), attb)
    zn2 = _dot_nt((ze * ze).astype(bf16), attb)
    inv_xn = jax.lax.rsqrt(jnp.maximum(xn2, 1e-24)) * (1.0 / HEADS)
    inv_zn = jax.lax.rsqrt(jnp.maximum(zn2, 1e-24))
    acc = jnp.zeros(x1f.shape[:1] + (E,), f32)
    for hh in range(HEADS):
        x1h = (x1f * inv_xn[:, hh:hh + 1]).astype(bf16)
        zth = (ze * attp[hh:hh + 1, :] * inv_zn[:, hh:hh + 1]
               ).astype(bf16)
        acc += _dot_nt(x1h, zth)
    ac = jnp.minimum(jnp.maximum(acc, 1e-6), 1.0 - 1e-6)
    return ac * jnp.log(ac * 2.0) + (1.0 - ac) * jnp.log((1.0 - ac) * 2.0)


def _p3(x1_ref, hb_ref, ze_ref, att_ref, w11_ref, b11_ref, w2_ref, b2_ref,
        kl_ref, t1_ref, s2_ref):
    i = pl.program_id(0)

    @pl.when(i == 0)
    def _():
        kl_ref[...] = jnp.zeros_like(kl_ref)
        t1_ref[...] = jnp.zeros_like(t1_ref)
        s2_ref[...] = jnp.zeros_like(s2_ref)

    x1 = x1_ref[...]
    hb = hb_ref[...]
    kl = _attn_kl(x1.astype(f32), ze_ref[...], att_ref)
    kl_ref[...] += jnp.sum(kl, axis=0, keepdims=True)
    t1_ref[...] += _dot_tn(hb, (_dot(x1, w11_ref[...]) +
                                b11_ref[...]).astype(bf16))
    s2_ref[...] += _dot_tn(hb, (_dot(x1, w2_ref[...]) +
                                b2_ref[...]).astype(bf16))


def _p4(hb_ref, y2_ref, u1_ref, x2_ref, y1_ref, ze2_ref):
    i = pl.program_id(0)

    @pl.when(i == 0)
    def _():
        ze2_ref[...] = jnp.zeros_like(ze2_ref)

    hb = hb_ref[...]
    dv = jnp.maximum(jnp.sum(hb.astype(f32), axis=1, keepdims=True), 1.0)
    x2 = jnp.maximum(_dot(hb, y2_ref[...]) / dv, 0.0).astype(bf16)
    x2_ref[...] = x2
    y1_ref[...] = _dot(hb, u1_ref[...]) / dv
    ze2_ref[...] += _dot_tn(hb, x2)


def _p5(x2_ref, hb_ref, ze_ref, att_ref, w21_ref, b21_ref, kl_ref, t2_ref):
    i = pl.program_id(0)

    @pl.when(i == 0)
    def _():
        kl_ref[...] = jnp.zeros_like(kl_ref)
        t2_ref[...] = jnp.zeros_like(t2_ref)

    x2 = x2_ref[...]
    hb = hb_ref[...]
    kl = _attn_kl(x2.astype(f32), ze_ref[...], att_ref)
    kl_ref[...] += jnp.sum(kl, axis=0, keepdims=True)
    t2_ref[...] += _dot_tn(hb, (_dot(x2, w21_ref[...]) +
                                b21_ref[...]).astype(bf16))


def _p6(hb_ref, u2_ref, y2_ref):
    hb = hb_ref[...]
    dv = jnp.maximum(jnp.sum(hb.astype(f32), axis=1, keepdims=True), 1.0)
    y2_ref[...] = _dot(hb, u2_ref[...]) / dv


def _row_spec(cols):
    return pl.BlockSpec((TILE, cols), lambda i: (i, 0))


def _full_spec(rows, cols):
    return pl.BlockSpec((rows, cols), lambda i: (0, 0))


def kernel(x, H, W1, b1, att1, W11, b11, W2, b2, att2, W21, b21):
    b1r = b1.reshape(1, C)
    b2r = b2.reshape(1, C)
    att1sq = jnp.zeros((NCP, C), f32).at[:HEADS].set(att1 * att1)
    att2sq = jnp.zeros((NCP, C), f32).at[:HEADS].set(att2 * att2)
    w11p = jnp.zeros((C, NCP), bf16).at[:, :3].set(W11.astype(bf16))
    b11p = jnp.zeros((1, NCP), f32).at[0, :3].set(b11)
    w21p = jnp.zeros((C, NCP), bf16).at[:, :3].set(W21.astype(bf16))
    b21p = jnp.zeros((1, NCP), f32).at[0, :3].set(b21)

    s1, de, hb = pl.pallas_call(
        _p1,
        grid=(GRID,),
        in_specs=[_row_spec(C), _row_spec(E), _full_spec(C, C),
                  _full_spec(1, C)],
        out_specs=[_full_spec(E, C), _full_spec(1, E), _row_spec(E)],
        out_shape=[jax.ShapeDtypeStruct((E, C), f32),
                   jax.ShapeDtypeStruct((1, E), f32),
                   jax.ShapeDtypeStruct((N, E), bf16)],
        compiler_params=_ARB,
    )(x, H, W1.astype(bf16), b1r)

    de_col = jnp.maximum(de, 1.0).reshape(E, 1)

    x1, ze1 = pl.pallas_call(
        _p2,
        grid=(GRID,),
        in_specs=[_row_spec(E), _full_spec(E, C)],
        out_specs=[_row_spec(C), _full_spec(E, C)],
        out_shape=[jax.ShapeDtypeStruct((N, C), bf16),
                   jax.ShapeDtypeStruct((E, C), f32)],
        compiler_params=_ARB,
    )(hb, (s1 / de_col).astype(bf16))

    klv1, t1, s2 = pl.pallas_call(
        _p3,
        grid=(GRID,),
        in_specs=[_row_spec(C), _row_spec(E), _full_spec(E, C),
                  _full_spec(NCP, C), _full_spec(C, NCP), _full_spec(1, NCP),
                  _full_spec(C, C), _full_spec(1, C)],
        out_specs=[_full_spec(1, E), _full_spec(E, NCP), _full_spec(E, C)],
        out_shape=[jax.ShapeDtypeStruct((1, E), f32),
                   jax.ShapeDtypeStruct((E, NCP), f32),
                   jax.ShapeDtypeStruct((E, C), f32)],
        compiler_params=_ARB,
    )(x1, hb, ze1, att1sq, w11p, b11p, W2.astype(bf16), b2r)

    x2, y1p, ze2 = pl.pallas_call(
        _p4,
        grid=(GRID,),
        in_specs=[_row_spec(E), _full_spec(E, C), _full_spec(E, NCP)],
        out_specs=[_row_spec(C), _row_spec(NCP), _full_spec(E, C)],
        out_shape=[jax.ShapeDtypeStruct((N, C), bf16),
                   jax.ShapeDtypeStruct((N, NCP), f32),
                   jax.ShapeDtypeStruct((E, C), f32)],
        compiler_params=_ARB,
    )(hb, (s2 / de_col).astype(bf16), (t1 / de_col).astype(bf16))

    klv2, t2 = pl.pallas_call(
        _p5,
        grid=(GRID,),
        in_specs=[_row_spec(C), _row_spec(E), _full_spec(E, C),
                  _full_spec(NCP, C), _full_spec(C, NCP), _full_spec(1, NCP)],
        out_specs=[_full_spec(1, E), _full_spec(E, NCP)],
        out_shape=[jax.ShapeDtypeStruct((1, E), f32),
                   jax.ShapeDtypeStruct((E, NCP), f32)],
        compiler_params=_ARB,
    )(x2, hb, ze2, att2sq, w21p, b21p)

    y2p = pl.pallas_call(
        _p6,
        grid=(GRID,),
        in_specs=[_row_spec(E), _full_spec(E, NCP)],
        out_specs=_row_spec(NCP),
        out_shape=jax.ShapeDtypeStruct((N, NCP), f32),
        compiler_params=_ARB,
    )(hb, (t2 / de_col).astype(bf16))

    kl1 = jnp.sum(klv1) / N
    kl2 = jnp.sum(klv2) / N
    return (y1p[:, :3], y2p[:, :3], (kl1 + kl2) * 0.5)


# speedup vs baseline: 1.5191x; 1.1501x over previous
import jax
import jax.numpy as jnp
from jax.experimental import pallas as pl
from jax.experimental.pallas import tpu as pltpu

N = 10000
E = 256
C = 256
HEADS = 8
TILE = 1000
GRID = N // TILE
NCP = 128

f32 = jnp.float32
bf16 = jnp.bfloat16


def _dot(a, b):
    return jax.lax.dot_general(a, b, (((1,), (0,)), ((), ())),
                               preferred_element_type=f32)


def _dot_tn(a, b):
    return jax.lax.dot_general(a, b, (((0,), (0,)), ((), ())),
                               preferred_element_type=f32)


def _dot_nt(a, b):
    return jax.lax.dot_general(a, b, (((1,), (1,)), ((), ())),
                               preferred_element_type=f32)


def _mega(x_ref, h_ref, w1_ref, b1_ref, att1_ref, w11_ref, b11_ref,
          w2_ref, b2_ref, att2_ref, w21_ref, b21_ref,
          klv1_ref, klv2_ref, y1_ref, y2_ref,
          hb_s, x1_s, x2_s, s_s, ze_s, t_s, de_s, dec_s, yb_s, ub_s, zth_s):
    p = pl.program_id(0)
    i = pl.program_id(1)
    rows = pl.ds(i * TILE, TILE)

    def _rowsum_deg(hb):
        return jnp.maximum(
            jnp.sum(hb.astype(f32), axis=1, keepdims=True), 1.0)

    def _finalize_edge(ze, att_ref):
        attp = att_ref[...]
        zn2 = _dot_nt((ze * ze).astype(bf16), attp.astype(bf16))
        inv_zn = jax.lax.rsqrt(jnp.maximum(zn2, 1e-24))
        for hh in range(HEADS):
            zth_s[:, hh * C:(hh + 1) * C] = (
                ze * attp[hh:hh + 1, :] * inv_zn[:, hh:hh + 1]).astype(bf16)

    def _attn_kl(xf, att_ref):
        attb = att_ref[...].astype(bf16)
        xn2 = _dot_nt((xf * xf).astype(bf16), attb)
        inv_xn = jax.lax.rsqrt(jnp.maximum(xn2, 1e-24)) * (1.0 / HEADS)
        acc = jnp.zeros((TILE, E), f32)
        for hh in range(HEADS):
            xh = (xf * inv_xn[:, hh:hh + 1]).astype(bf16)
            acc += _dot_nt(xh, zth_s[:, hh * C:(hh + 1) * C])
        ac = jnp.minimum(jnp.maximum(acc, 1e-6), 1.0 - 1e-6)
        kl = ac * jnp.log(ac * 2.0) + (1.0 - ac) * jnp.log((1.0 - ac) * 2.0)
        return jnp.sum(kl, axis=0, keepdims=True)

    @pl.when(jnp.logical_and(p == 0, i == 0))
    def _():
        s_s[...] = jnp.zeros_like(s_s)
        de_s[...] = jnp.zeros_like(de_s)
        klv1_ref[...] = jnp.zeros_like(klv1_ref)
        klv2_ref[...] = jnp.zeros_like(klv2_ref)

    @pl.when(p == 0)
    def _():
        h = h_ref[...]
        hb = h.astype(bf16)
        hb_s[rows, :] = hb
        xw = _dot(x_ref[...].astype(bf16), w1_ref[...]) + b1_ref[...]
        s_s[...] += _dot_tn(hb, xw.astype(bf16))
        de_s[...] += jnp.sum(h, axis=0, keepdims=True)

    @pl.when(jnp.logical_and(p == 0, i == GRID - 1))
    def _():
        dec = jnp.maximum(de_s[...], 1.0).reshape(E, 1)
        dec_s[...] = dec
        yb_s[...] = (s_s[...] / dec).astype(bf16)

    @pl.when(jnp.logical_and(p == 1, i == 0))
    def _():
        ze_s[...] = jnp.zeros_like(ze_s)

    @pl.when(p == 1)
    def _():
        hb = hb_s[rows, :]
        dv = _rowsum_deg(hb)
        x1 = jnp.maximum(_dot(hb, yb_s[...]) / dv, 0.0).astype(bf16)
        x1_s[rows, :] = x1
        ze_s[...] += _dot_tn(hb, x1)

    @pl.when(jnp.logical_and(p == 1, i == GRID - 1))
    def _():
        _finalize_edge(ze_s[...], att1_ref)

    @pl.when(jnp.logical_and(p == 2, i == 0))
    def _():
        s_s[...] = jnp.zeros_like(s_s)
        t_s[...] = jnp.zeros_like(t_s)

    @pl.when(p == 2)
    def _():
        x1 = x1_s[rows, :]
        hb = hb_s[rows, :]
        klv1_ref[...] += _attn_kl(x1.astype(f32), att1_ref)
        t_s[...] += _dot_tn(hb, (_dot(x1, w11_ref[...]) +
                                 b11_ref[...]).astype(bf16))
        s_s[...] += _dot_tn(hb, (_dot(x1, w2_ref[...]) +
                                 b2_ref[...]).astype(bf16))

    @pl.when(jnp.logical_and(p == 2, i == GRID - 1))
    def _():
        dec = dec_s[...]
        yb_s[...] = (s_s[...] / dec).astype(bf16)
        ub_s[...] = (t_s[...] / dec).astype(bf16)

    @pl.when(jnp.logical_and(p == 3, i == 0))
    def _():
        ze_s[...] = jnp.zeros_like(ze_s)

    @pl.when(p == 3)
    def _():
        hb = hb_s[rows, :]
        dv = _rowsum_deg(hb)
        x2 = jnp.maximum(_dot(hb, yb_s[...]) / dv, 0.0).astype(bf16)
        x2_s[rows, :] = x2
        y1_ref[...] = _dot(hb, ub_s[...]) / dv
        ze_s[...] += _dot_tn(hb, x2)

    @pl.when(jnp.logical_and(p == 3, i == GRID - 1))
    def _():
        _finalize_edge(ze_s[...], att2_ref)

    @pl.when(jnp.logical_and(p == 4, i == 0))
    def _():
        t_s[...] = jnp.zeros_like(t_s)

    @pl.when(p == 4)
    def _():
        x2 = x2_s[rows, :]
        hb = hb_s[rows, :]
        klv2_ref[...] += _attn_kl(x2.astype(f32), att2_ref)
        t_s[...] += _dot_tn(hb, (_dot(x2, w21_ref[...]) +
                                 b21_ref[...]).astype(bf16))

    @pl.when(jnp.logical_and(p == 4, i == GRID - 1))
    def _():
        ub_s[...] = (t_s[...] / dec_s[...]).astype(bf16)

    @pl.when(p == 5)
    def _():
        hb = hb_s[rows, :]
        dv = _rowsum_deg(hb)
        y2_ref[...] = _dot(hb, ub_s[...]) / dv


def _stream_spec(cols, phase):
    def idx(p, i):
        return (jnp.where(p == phase, i, jnp.where(p < phase, 0, GRID - 1)),
                0)
    return pl.BlockSpec((TILE, cols), idx)


def _const_spec(rows, cols):
    return pl.BlockSpec((rows, cols), lambda p, i: (0, 0))


def kernel(x, H, W1, b1, att1, W11, b11, W2, b2, att2, W21, b21):
    b1r = b1.reshape(1, C)
    b2r = b2.reshape(1, C)
    att1sq = jnp.zeros((NCP, C), f32).at[:HEADS].set(att1 * att1)
    att2sq = jnp.zeros((NCP, C), f32).at[:HEADS].set(att2 * att2)
    w11p = jnp.zeros((C, NCP), bf16).at[:, :3].set(W11.astype(bf16))
    b11p = jnp.zeros((1, NCP), f32).at[0, :3].set(b11)
    w21p = jnp.zeros((C, NCP), bf16).at[:, :3].set(W21.astype(bf16))
    b21p = jnp.zeros((1, NCP), f32).at[0, :3].set(b21)

    klv1, klv2, y1p, y2p = pl.pallas_call(
        _mega,
        grid=(6, GRID),
        in_specs=[
            _stream_spec(C, 0),
            _stream_spec(E, 0),
            _const_spec(C, C),
            _const_spec(1, C),
            _const_spec(NCP, C),
            _const_spec(C, NCP),
            _const_spec(1, NCP),
            _const_spec(C, C),
            _const_spec(1, C),
            _const_spec(NCP, C),
            _const_spec(C, NCP),
            _const_spec(1, NCP),
        ],
        out_specs=[
            _const_spec(1, E),
            _const_spec(1, E),
            _stream_spec(NCP, 3),
            _stream_spec(NCP, 5),
        ],
        out_shape=[
            jax.ShapeDtypeStruct((1, E), f32),
            jax.ShapeDtypeStruct((1, E), f32),
            jax.ShapeDtypeStruct((N, NCP), f32),
            jax.ShapeDtypeStruct((N, NCP), f32),
        ],
        scratch_shapes=[
            pltpu.VMEM((N, E), bf16),
            pltpu.VMEM((N, C), bf16),
            pltpu.VMEM((N, C), bf16),
            pltpu.VMEM((E, C), f32),
            pltpu.VMEM((E, C), f32),
            pltpu.VMEM((E, NCP), f32),
            pltpu.VMEM((1, E), f32),
            pltpu.VMEM((E, 1), f32),
            pltpu.VMEM((E, C), bf16),
            pltpu.VMEM((E, NCP), bf16),
            pltpu.VMEM((E, HEADS * C), bf16),
        ],
        compiler_params=pltpu.CompilerParams(
            dimension_semantics=("arbitrary", "arbitrary")),
    )(x, H, W1.astype(bf16), b1r, att1sq, w11p, b11p,
      W2.astype(bf16), b2r, att2sq, w21p, b21p)

    kl1 = jnp.sum(klv1) / N
    kl2 = jnp.sum(klv2) / N
    return (y1p[:, :3], y2p[:, :3], (kl1 + kl2) * 0.5)


# speedup vs baseline: 1.7128x; 1.1275x over previous
import jax
import jax.numpy as jnp
from jax.experimental import pallas as pl
from jax.experimental.pallas import tpu as pltpu

N = 10000
E = 256
C = 256
HEADS = 8
TILE = 2000
GRID = N // TILE
NCP = 128

f32 = jnp.float32
bf16 = jnp.bfloat16


def _dot(a, b):
    return jax.lax.dot_general(a, b, (((1,), (0,)), ((), ())),
                               preferred_element_type=f32)


def _dot_tn(a, b):
    return jax.lax.dot_general(a, b, (((0,), (0,)), ((), ())),
                               preferred_element_type=f32)


def _dot_nt(a, b):
    return jax.lax.dot_general(a, b, (((1,), (1,)), ((), ())),
                               preferred_element_type=f32)


def _mega(x_ref, h_ref, w1_ref, b1_ref, att1_ref, w11_ref, b11_ref,
          w2_ref, b2_ref, att2_ref, w21_ref, b21_ref,
          klv1_ref, klv2_ref, y1_ref, y2_ref,
          hb_s, x1_s, x2_s, s_s, ze_s, t_s, de_s, dec_s, yb_s, ub_s, zth_s):
    p = pl.program_id(0)
    i = pl.program_id(1)
    rows = pl.ds(i * TILE, TILE)

    def _rowsum_deg(hb):
        return jnp.maximum(
            jnp.sum(hb.astype(f32), axis=1, keepdims=True), 1.0)

    def _finalize_edge(ze, att_ref):
        attp = att_ref[...]
        zn2 = _dot_nt((ze * ze).astype(bf16), attp.astype(bf16))
        inv_zn = jax.lax.rsqrt(jnp.maximum(zn2, 1e-24))
        for hh in range(HEADS):
            zth_s[:, hh * C:(hh + 1) * C] = (
                ze * attp[hh:hh + 1, :] * inv_zn[:, hh:hh + 1]).astype(bf16)

    def _attn_kl(xf, att_ref):
        attb = att_ref[...].astype(bf16)
        xn2 = _dot_nt((xf * xf).astype(bf16), attb)
        inv_xn = jax.lax.rsqrt(jnp.maximum(xn2, 1e-24)) * (1.0 / HEADS)
        acc = jnp.zeros((TILE, E), f32)
        for hh in range(HEADS):
            xh = (xf * inv_xn[:, hh:hh + 1]).astype(bf16)
            acc += _dot_nt(xh, zth_s[:, hh * C:(hh + 1) * C])
        ac = jnp.minimum(jnp.maximum(acc, 1e-6), 1.0 - 1e-6)
        kl = ac * jnp.log(ac * 2.0) + (1.0 - ac) * jnp.log((1.0 - ac) * 2.0)
        return jnp.sum(kl, axis=0, keepdims=True)

    @pl.when(jnp.logical_and(p == 0, i == 0))
    def _():
        s_s[...] = jnp.zeros_like(s_s)
        de_s[...] = jnp.zeros_like(de_s)
        klv1_ref[...] = jnp.zeros_like(klv1_ref)
        klv2_ref[...] = jnp.zeros_like(klv2_ref)

    @pl.when(p == 0)
    def _():
        h = h_ref[...]
        hb = h.astype(bf16)
        hb_s[rows, :] = hb
        xw = _dot(x_ref[...].astype(bf16), w1_ref[...]) + b1_ref[...]
        s_s[...] += _dot_tn(hb, xw.astype(bf16))
        de_s[...] += jnp.sum(h, axis=0, keepdims=True)

    @pl.when(jnp.logical_and(p == 0, i == GRID - 1))
    def _():
        dec = jnp.maximum(de_s[...], 1.0).reshape(E, 1)
        dec_s[...] = dec
        yb_s[...] = (s_s[...] / dec).astype(bf16)

    @pl.when(jnp.logical_and(p == 1, i == 0))
    def _():
        ze_s[...] = jnp.zeros_like(ze_s)

    @pl.when(p == 1)
    def _():
        hb = hb_s[rows, :]
        dv = _rowsum_deg(hb)
        x1 = jnp.maximum(_dot(hb, yb_s[...]) / dv, 0.0).astype(bf16)
        x1_s[rows, :] = x1
        ze_s[...] += _dot_tn(hb, x1)

    @pl.when(jnp.logical_and(p == 1, i == GRID - 1))
    def _():
        _finalize_edge(ze_s[...], att1_ref)

    @pl.when(jnp.logical_and(p == 2, i == 0))
    def _():
        s_s[...] = jnp.zeros_like(s_s)
        t_s[...] = jnp.zeros_like(t_s)

    @pl.when(p == 2)
    def _():
        x1 = x1_s[rows, :]
        hb = hb_s[rows, :]
        klv1_ref[...] += _attn_kl(x1.astype(f32), att1_ref)
        t_s[...] += _dot_tn(hb, (_dot(x1, w11_ref[...]) +
                                 b11_ref[...]).astype(bf16))
        s_s[...] += _dot_tn(hb, (_dot(x1, w2_ref[...]) +
                                 b2_ref[...]).astype(bf16))

    @pl.when(jnp.logical_and(p == 2, i == GRID - 1))
    def _():
        dec = dec_s[...]
        yb_s[...] = (s_s[...] / dec).astype(bf16)
        ub_s[...] = (t_s[...] / dec).astype(bf16)

    @pl.when(jnp.logical_and(p == 3, i == 0))
    def _():
        ze_s[...] = jnp.zeros_like(ze_s)

    @pl.when(p == 3)
    def _():
        hb = hb_s[rows, :]
        dv = _rowsum_deg(hb)
        x2 = jnp.maximum(_dot(hb, yb_s[...]) / dv, 0.0).astype(bf16)
        x2_s[rows, :] = x2
        y1_ref[...] = _dot(hb, ub_s[...]) / dv
        ze_s[...] += _dot_tn(hb, x2)

    @pl.when(jnp.logical_and(p == 3, i == GRID - 1))
    def _():
        _finalize_edge(ze_s[...], att2_ref)

    @pl.when(jnp.logical_and(p == 4, i == 0))
    def _():
        t_s[...] = jnp.zeros_like(t_s)

    @pl.when(p == 4)
    def _():
        x2 = x2_s[rows, :]
        hb = hb_s[rows, :]
        klv2_ref[...] += _attn_kl(x2.astype(f32), att2_ref)
        t_s[...] += _dot_tn(hb, (_dot(x2, w21_ref[...]) +
                                 b21_ref[...]).astype(bf16))

    @pl.when(jnp.logical_and(p == 4, i == GRID - 1))
    def _():
        ub_s[...] = (t_s[...] / dec_s[...]).astype(bf16)

    @pl.when(p == 5)
    def _():
        hb = hb_s[rows, :]
        dv = _rowsum_deg(hb)
        y2_ref[...] = _dot(hb, ub_s[...]) / dv


def _stream_spec(cols, phase):
    def idx(p, i):
        return (jnp.where(p == phase, i, jnp.where(p < phase, 0, GRID - 1)),
                0)
    return pl.BlockSpec((TILE, cols), idx)


def _const_spec(rows, cols):
    return pl.BlockSpec((rows, cols), lambda p, i: (0, 0))


def kernel(x, H, W1, b1, att1, W11, b11, W2, b2, att2, W21, b21):
    b1r = b1.reshape(1, C)
    b2r = b2.reshape(1, C)
    att1sq = jnp.zeros((NCP, C), f32).at[:HEADS].set(att1 * att1)
    att2sq = jnp.zeros((NCP, C), f32).at[:HEADS].set(att2 * att2)
    w11p = jnp.zeros((C, NCP), bf16).at[:, :3].set(W11.astype(bf16))
    b11p = jnp.zeros((1, NCP), f32).at[0, :3].set(b11)
    w21p = jnp.zeros((C, NCP), bf16).at[:, :3].set(W21.astype(bf16))
    b21p = jnp.zeros((1, NCP), f32).at[0, :3].set(b21)

    klv1, klv2, y1p, y2p = pl.pallas_call(
        _mega,
        grid=(6, GRID),
        in_specs=[
            _stream_spec(C, 0),
            _stream_spec(E, 0),
            _const_spec(C, C),
            _const_spec(1, C),
            _const_spec(NCP, C),
            _const_spec(C, NCP),
            _const_spec(1, NCP),
            _const_spec(C, C),
            _const_spec(1, C),
            _const_spec(NCP, C),
            _const_spec(C, NCP),
            _const_spec(1, NCP),
        ],
        out_specs=[
            _const_spec(1, E),
            _const_spec(1, E),
            _stream_spec(NCP, 3),
            _stream_spec(NCP, 5),
        ],
        out_shape=[
            jax.ShapeDtypeStruct((1, E), f32),
            jax.ShapeDtypeStruct((1, E), f32),
            jax.ShapeDtypeStruct((N, NCP), f32),
            jax.ShapeDtypeStruct((N, NCP), f32),
        ],
        scratch_shapes=[
            pltpu.VMEM((N, E), bf16),
            pltpu.VMEM((N, C), bf16),
            pltpu.VMEM((N, C), bf16),
            pltpu.VMEM((E, C), f32),
            pltpu.VMEM((E, C), f32),
            pltpu.VMEM((E, NCP), f32),
            pltpu.VMEM((1, E), f32),
            pltpu.VMEM((E, 1), f32),
            pltpu.VMEM((E, C), bf16),
            pltpu.VMEM((E, NCP), bf16),
            pltpu.VMEM((E, HEADS * C), bf16),
        ],
        compiler_params=pltpu.CompilerParams(
            dimension_semantics=("arbitrary", "arbitrary")),
    )(x, H, W1.astype(bf16), b1r, att1sq, w11p, b11p,
      W2.astype(bf16), b2r, att2sq, w21p, b21p)

    kl1 = jnp.sum(klv1) / N
    kl2 = jnp.sum(klv2) / N
    return (y1p[:, :3], y2p[:, :3], (kl1 + kl2) * 0.5)


# speedup vs baseline: 1.7197x; 1.0040x over previous
import jax
import jax.numpy as jnp
from jax.experimental import pallas as pl
from jax.experimental.pallas import tpu as pltpu

N = 10000
E = 256
C = 256
HEADS = 8
TILE = 2000
GRID = N // TILE
NCP = 128

f32 = jnp.float32
bf16 = jnp.bfloat16


def _dot(a, b):
    return jax.lax.dot_general(a, b, (((1,), (0,)), ((), ())),
                               preferred_element_type=f32)


def _dot_tn(a, b):
    return jax.lax.dot_general(a, b, (((0,), (0,)), ((), ())),
                               preferred_element_type=f32)


def _mega(x_ref, h_ref, w1_ref, b1_ref, att1_ref, att1t_ref, w11_ref,
          b11_ref, w2_ref, b2_ref, att2_ref, att2t_ref, w21_ref, b21_ref,
          klv1_ref, klv2_ref, y1_ref, y2_ref,
          hb_s, x1_s, x2_s, s_s, ze_s, t_s, de_s, dec_s, idv_s, yb_s, ub_s,
          zth_s):
    p = pl.program_id(0)
    i = pl.program_id(1)
    rows = pl.ds(i * TILE, TILE)

    def _finalize_edge(ze, att_ref):
        attp = att_ref[...]
        zn2 = _dot(ze * ze, jnp.transpose(attp))
        inv_zn = jax.lax.rsqrt(jnp.maximum(zn2, 1e-24))
        for hh in range(HEADS):
            zth_s[hh * C:(hh + 1) * C, :] = jnp.transpose(
                (ze * attp[hh:hh + 1, :] * inv_zn[:, hh:hh + 1])
            ).astype(bf16)

    def _attn_kl(xb, attt_ref):
        xn2 = _dot(xb * xb, attt_ref[...])
        inv_xn = (jax.lax.rsqrt(jnp.maximum(xn2, 1e-24)) *
                  (1.0 / HEADS)).astype(bf16)
        acc = jnp.zeros((TILE, E), f32)
        for hh in range(HEADS):
            acc += _dot(xb * inv_xn[:, hh:hh + 1],
                        zth_s[hh * C:(hh + 1) * C, :])
        ac = jnp.minimum(jnp.maximum(acc, 1e-6), 1.0 - 1e-6)
        kl = ac * jnp.log(ac * 2.0) + (1.0 - ac) * jnp.log((1.0 - ac) * 2.0)
        return jnp.sum(kl, axis=0, keepdims=True)

    @pl.when(jnp.logical_and(p == 0, i == 0))
    def _():
        s_s[...] = jnp.zeros_like(s_s)
        de_s[...] = jnp.zeros_like(de_s)
        klv1_ref[...] = jnp.zeros_like(klv1_ref)
        klv2_ref[...] = jnp.zeros_like(klv2_ref)

    @pl.when(p == 0)
    def _():
        h = h_ref[...]
        hb = h.astype(bf16)
        hb_s[rows, :] = hb
        idv_s[rows, :] = 1.0 / jnp.maximum(
            jnp.sum(h, axis=1, keepdims=True), 1.0)
        xw = _dot(x_ref[...].astype(bf16), w1_ref[...]) + b1_ref[...]
        s_s[...] += _dot_tn(hb, xw.astype(bf16))
        de_s[...] += jnp.sum(h, axis=0, keepdims=True)

    @pl.when(jnp.logical_and(p == 0, i == GRID - 1))
    def _():
        dec = jnp.maximum(de_s[...], 1.0).reshape(E, 1)
        dec_s[...] = dec
        yb_s[...] = (s_s[...] / dec).astype(bf16)

    @pl.when(jnp.logical_and(p == 1, i == 0))
    def _():
        ze_s[...] = jnp.zeros_like(ze_s)

    @pl.when(p == 1)
    def _():
        hb = hb_s[rows, :]
        x1 = jnp.maximum(_dot(hb, yb_s[...]) * idv_s[rows, :],
                         0.0).astype(bf16)
        x1_s[rows, :] = x1
        ze_s[...] += _dot_tn(hb, x1)

    @pl.when(jnp.logical_and(p == 1, i == GRID - 1))
    def _():
        _finalize_edge(ze_s[...], att1_ref)

    @pl.when(jnp.logical_and(p == 2, i == 0))
    def _():
        s_s[...] = jnp.zeros_like(s_s)
        t_s[...] = jnp.zeros_like(t_s)

    @pl.when(p == 2)
    def _():
        x1 = x1_s[rows, :]
        hb = hb_s[rows, :]
        klv1_ref[...] += _attn_kl(x1, att1t_ref)
        t_s[...] += _dot_tn(hb, (_dot(x1, w11_ref[...]) +
                                 b11_ref[...]).astype(bf16))
        s_s[...] += _dot_tn(hb, (_dot(x1, w2_ref[...]) +
                                 b2_ref[...]).astype(bf16))

    @pl.when(jnp.logical_and(p == 2, i == GRID - 1))
    def _():
        dec = dec_s[...]
        yb_s[...] = (s_s[...] / dec).astype(bf16)
        ub_s[...] = (t_s[...] / dec).astype(bf16)

    @pl.when(jnp.logical_and(p == 3, i == 0))
    def _():
        ze_s[...] = jnp.zeros_like(ze_s)

    @pl.when(p == 3)
    def _():
        hb = hb_s[rows, :]
        idv = idv_s[rows, :]
        x2 = jnp.maximum(_dot(hb, yb_s[...]) * idv, 0.0).astype(bf16)
        x2_s[rows, :] = x2
        y1_ref[...] = _dot(hb, ub_s[...]) * idv
        ze_s[...] += _dot_tn(hb, x2)

    @pl.when(jnp.logical_and(p == 3, i == GRID - 1))
    def _():
        _finalize_edge(ze_s[...], att2_ref)

    @pl.when(jnp.logical_and(p == 4, i == 0))
    def _():
        t_s[...] = jnp.zeros_like(t_s)

    @pl.when(p == 4)
    def _():
        x2 = x2_s[rows, :]
        hb = hb_s[rows, :]
        klv2_ref[...] += _attn_kl(x2, att2t_ref)
        t_s[...] += _dot_tn(hb, (_dot(x2, w21_ref[...]) +
                                 b21_ref[...]).astype(bf16))

    @pl.when(jnp.logical_and(p == 4, i == GRID - 1))
    def _():
        ub_s[...] = (t_s[...] / dec_s[...]).astype(bf16)

    @pl.when(p == 5)
    def _():
        hb = hb_s[rows, :]
        y2_ref[...] = _dot(hb, ub_s[...]) * idv_s[rows, :]


def _stream_spec(cols, phase):
    def idx(p, i):
        return (jnp.where(p == phase, i, jnp.where(p < phase, 0, GRID - 1)),
                0)
    return pl.BlockSpec((TILE, cols), idx)


def _const_spec(rows, cols):
    return pl.BlockSpec((rows, cols), lambda p, i: (0, 0))


def kernel(x, H, W1, b1, att1, W11, b11, W2, b2, att2, W21, b21):
    b1r = b1.reshape(1, C)
    b2r = b2.reshape(1, C)
    att1sq = jnp.zeros((NCP, C), f32).at[:HEADS].set(att1 * att1)
    att2sq = jnp.zeros((NCP, C), f32).at[:HEADS].set(att2 * att2)
    att1sqT = att1sq.T.astype(bf16)
    att2sqT = att2sq.T.astype(bf16)
    w11p = jnp.zeros((C, NCP), bf16).at[:, :3].set(W11.astype(bf16))
    b11p = jnp.zeros((1, NCP), f32).at[0, :3].set(b11)
    w21p = jnp.zeros((C, NCP), bf16).at[:, :3].set(W21.astype(bf16))
    b21p = jnp.zeros((1, NCP), f32).at[0, :3].set(b21)

    klv1, klv2, y1p, y2p = pl.pallas_call(
        _mega,
        grid=(6, GRID),
        in_specs=[
            _stream_spec(C, 0),
            _stream_spec(E, 0),
            _const_spec(C, C),
            _const_spec(1, C),
            _const_spec(NCP, C),
            _const_spec(C, NCP),
            _const_spec(C, NCP),
            _const_spec(1, NCP),
            _const_spec(C, C),
            _const_spec(1, C),
            _const_spec(NCP, C),
            _const_spec(C, NCP),
            _const_spec(C, NCP),
            _const_spec(1, NCP),
        ],
        out_specs=[
            _const_spec(1, E),
            _const_spec(1, E),
            _stream_spec(NCP, 3),
            _stream_spec(NCP, 5),
        ],
        out_shape=[
            jax.ShapeDtypeStruct((1, E), f32),
            jax.ShapeDtypeStruct((1, E), f32),
            jax.ShapeDtypeStruct((N, NCP), f32),
            jax.ShapeDtypeStruct((N, NCP), f32),
        ],
        scratch_shapes=[
            pltpu.VMEM((N, E), bf16),
            pltpu.VMEM((N, C), bf16),
            pltpu.VMEM((N, C), bf16),
            pltpu.VMEM((E, C), f32),
            pltpu.VMEM((E, C), f32),
            pltpu.VMEM((E, NCP), f32),
            pltpu.VMEM((1, E), f32),
            pltpu.VMEM((E, 1), f32),
            pltpu.VMEM((N, 1), f32),
            pltpu.VMEM((E, C), bf16),
            pltpu.VMEM((E, NCP), bf16),
            pltpu.VMEM((HEADS * C, E), bf16),
        ],
        compiler_params=pltpu.CompilerParams(
            dimension_semantics=("arbitrary", "arbitrary")),
    )(x, H, W1.astype(bf16), b1r, att1sq, att1sqT, w11p, b11p,
      W2.astype(bf16), b2r, att2sq, att2sqT, w21p, b21p)

    kl1 = jnp.sum(klv1) / N
    kl2 = jnp.sum(klv2) / N
    return (y1p[:, :3], y2p[:, :3], (kl1 + kl2) * 0.5)
